# baseline (speedup 1.0000x reference)
import functools

import jax
import jax.numpy as jnp
from jax import lax
from jax.experimental import pallas as pl
from jax.experimental.pallas import tpu as pltpu, tpu_sc as plsc

_K = 16
_INF = float("inf")
_NLVL = 4


def _sel_body(posT_ref, pospad_ref, q_ref, idx_ref, ef8_ref, *, bsz, n, k):
    i = pl.program_id(0)
    q = q_ref[...]
    C = posT_ref.shape[1]

    dot = functools.partial(lax.dot_general,
                            preferred_element_type=jnp.float32)
    mm = lambda a, bb: dot(a, bb, (((1,), (0,)), ((), ())))

    pT = posT_ref[...]
    qsq = (q[:, 0:1] * q[:, 0:1] + q[:, 1:2] * q[:, 1:2]) + q[:, 2:3] * q[:, 2:3]
    psq = (pT[0:1, :] * pT[0:1, :] + pT[1:2, :] * pT[1:2, :]) + pT[2:3, :] * pT[2:3, :]
    qp = lax.dot_general(q.astype(jnp.bfloat16), pT.astype(jnp.bfloat16),
                         (((1,), (0,)), ((), ())),
                         preferred_element_type=jnp.float32)
    d2 = qsq - 2.0 * qp + psq
    rows_g = i * bsz + lax.broadcasted_iota(jnp.int32, (bsz, C), 0)
    cols = lax.broadcasted_iota(jnp.int32, (bsz, C), 1)
    d2 = jnp.where(cols == rows_g, _INF, d2)

    d2dir = ((q[:, 0:1] - pT[0:1, :]) ** 2
             + (q[:, 1:2] - pT[1:2, :]) ** 2
             + (q[:, 2:3] - pT[2:3, :]) ** 2)

    nslab = C // 128
    fv = [jnp.full((bsz, 128), _INF, jnp.float32) for _ in range(_NLVL)]
    fi = [jnp.zeros((bsz, 128), jnp.int32) for _ in range(_NLVL)]
    for s in range(nslab):
        xv = d2[:, s * 128:(s + 1) * 128]
        xi = jnp.full((bsz, 128), s, jnp.int32)
        for j in range(_NLVL):
            swap = xv < fv[j]
            fv[j], xv = jnp.where(swap, xv, fv[j]), jnp.where(swap, fv[j], xv)
            fi[j], xi = jnp.where(swap, xi, fi[j]), jnp.where(swap, fi[j], xi)
    sv = jnp.concatenate(fv, axis=1)
    si = jnp.concatenate(fi, axis=1)
    cols512 = lax.broadcasted_iota(jnp.int32, (bsz, 128 * _NLVL), 1)

    idx_parts = []
    t16 = jnp.zeros((bsz, 1), jnp.float32)
    for _ in range(k):
        t16 = jnp.min(sv, axis=1, keepdims=True)
        c = jnp.argmin(sv, axis=1).astype(jnp.int32).reshape(bsz, 1)
        sel = cols512 == c
        slab = jnp.sum(jnp.where(sel, si, 0), axis=1, keepdims=True)
        idx_parts.append(slab * 128 + jnp.remainder(c, 128))
        sv = jnp.where(sel, _INF, sv)
    idx_ref[...] = jnp.concatenate(idx_parts, axis=1)

    inv_k = jnp.float32(1.0 / k)
    mask = jnp.where(d2 <= t16, 1.0, 0.0)
    dmean = jnp.sum(mask * jnp.sqrt(d2dir), axis=1, keepdims=True) * inv_k
    rel8 = mm(mask, pospad_ref[...]) * inv_k - q
    col8 = lax.broadcasted_iota(jnp.int32, (bsz, 8), 1)
    ef8_ref[...] = jnp.where(col8 == 3, dmean, rel8)


def _make_sc_gather(NP, D, k):
    info = plsc.get_sparse_core_info()
    NC, NS, L = info.num_cores, info.num_subcores, info.num_lanes
    NW = NC * NS
    NR = NP // NW
    CH = 128 // k
    NG = NR // CH
    mesh = plsc.VectorSubcoreMesh(core_axis_name="c", subcore_axis_name="s")

    @functools.partial(
        pl.kernel, mesh=mesh,
        out_type=jax.ShapeDtypeStruct((NP, D), jnp.float32),
        scratch_types=[
            pltpu.VMEM((NR * k,), jnp.int32),
            pltpu.VMEM((CH * k, D), jnp.float32),
            pltpu.VMEM((CH, D), jnp.float32),
            pltpu.SemaphoreType.DMA,
        ],
    )
    def sc_gather(x_hbm, idx_hbm, out_hbm, idxv, rows, acc, sem):
        wid = lax.axis_index("s") * NC + lax.axis_index("c")
        pltpu.sync_copy(idx_hbm.at[pl.ds(wid * NR * k, NR * k)], idxv)

        def body(g, carry):
            pltpu.async_copy(
                x_hbm.at[idxv.at[pl.ds(g * CH * k, CH * k)]], rows, sem
            ).wait()
            for nn in range(CH):
                for l in range(D // L):
                    v = rows[nn * k, pl.ds(l * L, L)]
                    for j in range(1, k):
                        v = v + rows[nn * k + j, pl.ds(l * L, L)]
                    acc[nn, pl.ds(l * L, L)] = v
            pltpu.sync_copy(acc, out_hbm.at[pl.ds(wid * NR + g * CH, CH)])
            return carry

        lax.fori_loop(0, NG, body, 0)

    return sc_gather


def _dense_body(xblk_ref, xs_ref, ef_ref, w1_ref, w2_ref, wef_ref, prm_ref,
                out_ref, *, inv_k):
    dot = functools.partial(lax.dot_general,
                            preferred_element_type=jnp.float32)
    mm = lambda a, bb: dot(a, bb, (((1,), (0,)), ((), ())))
    h = (mm(xblk_ref[...], w1_ref[...])
         + mm(xs_ref[...] * inv_k, w2_ref[...])
         + mm(ef_ref[...], wef_ref[...])
         + prm_ref[0:1, :])
    mu = jnp.mean(h, axis=1, keepdims=True)
    hc = h - mu
    var = jnp.mean(hc * hc, axis=1, keepdims=True)
    h = hc / jnp.sqrt(var + 1e-5) * prm_ref[1:2, :] + prm_ref[2:3, :]
    out_ref[...] = jnp.maximum(h, 0.0)


def kernel(x, pos, W, b, gamma, beta):
    n, D = x.shape
    k = min(_K, n - 1)
    C = ((n + 127) // 128) * 128
    bsz = next(bb for bb in (80, 40, 16, 8, 4, 2, 1) if n % bb == 0)

    pospad = jnp.zeros((C, 8), jnp.float32)
    pospad = pospad.at[:n, :3].set(pos).at[n:, :3].set(1e4)
    posT8 = pospad.T
    W1T = W[:, :D].T
    W2T = W[:, D:2 * D].T
    Wef = jnp.zeros((8, D), jnp.float32).at[:3].set(W[:, 2 * D:2 * D + 3].T)
    Wef = Wef.at[3].set(W[:, 2 * D + 3])
    prm = jnp.zeros((8, D), jnp.float32)
    prm = prm.at[0].set(b).at[1].set(gamma).at[2].set(beta)

    grid = (n // bsz,)
    full = lambda shp: pl.BlockSpec(shp, lambda i: (0, 0))
    blk = lambda shp: pl.BlockSpec(shp, lambda i: (i, 0))

    idx, ef8 = pl.pallas_call(
        functools.partial(_sel_body, bsz=bsz, n=n, k=k),
        grid=grid,
        in_specs=[full((8, C)), full((C, 8)), blk((bsz, 8))],
        out_specs=[blk((bsz, k)), blk((bsz, 8))],
        out_shape=[jax.ShapeDtypeStruct((n, k), jnp.int32),
                   jax.ShapeDtypeStruct((n, 8), jnp.float32)],
    )(posT8, pospad, pospad)

    NP = ((n + 255) // 256) * 256
    idxf = jnp.zeros((NP * k,), jnp.int32).at[:n * k].set(idx.reshape(-1))
    xsum = _make_sc_gather(NP, D, k)(x, idxf)

    return pl.pallas_call(
        functools.partial(_dense_body, inv_k=1.0 / k),
        grid=grid,
        in_specs=[blk((bsz, D)), blk((bsz, D)), blk((bsz, 8)),
                  full((D, D)), full((D, D)), full((8, D)), full((8, D))],
        out_specs=blk((bsz, D)),
        out_shape=jax.ShapeDtypeStruct((n, D), jnp.float32),
    )(x, xsum, ef8, W1T, W2T, Wef, prm)


# speedup vs baseline: 1.0536x; 1.0536x over previous
import functools

import jax
import jax.numpy as jnp
from jax import lax
from jax.experimental import pallas as pl
from jax.experimental.pallas import tpu as pltpu, tpu_sc as plsc

_K = 16
_INF = float("inf")
_NLVL = 4


def _sel_body(posT_ref, pospad_ref, q_ref, idx_ref, ef8_ref, *, bsz, n, k):
    i = pl.program_id(0)
    q = q_ref[...]
    C = posT_ref.shape[1]

    dot = functools.partial(lax.dot_general,
                            preferred_element_type=jnp.float32)
    mm = lambda a, bb: dot(a, bb, (((1,), (0,)), ((), ())))

    pT = posT_ref[...]
    qsq = (q[:, 0:1] * q[:, 0:1] + q[:, 1:2] * q[:, 1:2]) + q[:, 2:3] * q[:, 2:3]
    psq = (pT[0:1, :] * pT[0:1, :] + pT[1:2, :] * pT[1:2, :]) + pT[2:3, :] * pT[2:3, :]
    qp = lax.dot_general(q.astype(jnp.bfloat16), pT.astype(jnp.bfloat16),
                         (((1,), (0,)), ((), ())),
                         preferred_element_type=jnp.float32)
    d2 = qsq - 2.0 * qp + psq
    rows_g = i * bsz + lax.broadcasted_iota(jnp.int32, (bsz, C), 0)
    cols = lax.broadcasted_iota(jnp.int32, (bsz, C), 1)
    d2 = jnp.where(cols == rows_g, _INF, d2)

    d2dir = ((q[:, 0:1] - pT[0:1, :]) ** 2
             + (q[:, 1:2] - pT[1:2, :]) ** 2
             + (q[:, 2:3] - pT[2:3, :]) ** 2)

    nslab = C // 128
    fv = [jnp.full((bsz, 128), _INF, jnp.float32) for _ in range(_NLVL)]
    fi = [jnp.zeros((bsz, 128), jnp.int32) for _ in range(_NLVL)]
    for s in range(nslab):
        xv = d2[:, s * 128:(s + 1) * 128]
        xi = jnp.full((bsz, 128), s, jnp.int32)
        for j in range(_NLVL):
            swap = xv < fv[j]
            fv[j], xv = jnp.where(swap, xv, fv[j]), jnp.where(swap, fv[j], xv)
            fi[j], xi = jnp.where(swap, xi, fi[j]), jnp.where(swap, fi[j], xi)
    sv = jnp.concatenate(fv, axis=1)
    si = jnp.concatenate(fi, axis=1)
    cols512 = lax.broadcasted_iota(jnp.int32, (bsz, 128 * _NLVL), 1)

    idx_parts = []
    t16 = jnp.zeros((bsz, 1), jnp.float32)
    for _ in range(k):
        t16 = jnp.min(sv, axis=1, keepdims=True)
        c = jnp.argmin(sv, axis=1).astype(jnp.int32).reshape(bsz, 1)
        sel = cols512 == c
        slab = jnp.sum(jnp.where(sel, si, 0), axis=1, keepdims=True)
        idx_parts.append(slab * 128 + jnp.remainder(c, 128))
        sv = jnp.where(sel, _INF, sv)
    idx_ref[...] = jnp.concatenate(idx_parts, axis=1)

    inv_k = jnp.float32(1.0 / k)
    mask = jnp.where(d2 <= t16, 1.0, 0.0)
    dmean = jnp.sum(mask * jnp.sqrt(d2dir), axis=1, keepdims=True) * inv_k
    rel8 = mm(mask, pospad_ref[...]) * inv_k - q
    col8 = lax.broadcasted_iota(jnp.int32, (bsz, 8), 1)
    ef8_ref[...] = jnp.where(col8 == 3, dmean, rel8)


def _make_sc_gather(NP, D, k):
    info = plsc.get_sparse_core_info()
    NC, NS, L = info.num_cores, info.num_subcores, info.num_lanes
    NW = NC * NS
    NR = NP // NW
    CH = 128 // k
    NG = NR // CH
    mesh = plsc.VectorSubcoreMesh(core_axis_name="c", subcore_axis_name="s")

    @functools.partial(
        pl.kernel, mesh=mesh,
        out_type=jax.ShapeDtypeStruct((NP, D), jnp.float32),
        scratch_types=[
            pltpu.VMEM((NR * k,), jnp.int32),
            pltpu.VMEM((CH * k, D), jnp.float32),
            pltpu.VMEM((CH * k, D), jnp.float32),
            pltpu.VMEM((CH, D), jnp.float32),
            pltpu.SemaphoreType.DMA,
            pltpu.SemaphoreType.DMA,
        ],
    )
    def sc_gather(x_hbm, idx_hbm, out_hbm, idxv, rows0, rows1, acc, s0, s1):
        wid = lax.axis_index("s") * NC + lax.axis_index("c")
        pltpu.sync_copy(idx_hbm.at[pl.ds(wid * NR * k, NR * k)], idxv)

        def gather(g, buf, sem):
            pltpu.async_copy(
                x_hbm.at[idxv.at[pl.ds(g * CH * k, CH * k)]], buf, sem)

        def drain(buf, sem):
            pltpu.make_async_copy(x_hbm.at[pl.ds(0, CH * k)], buf, sem).wait()

        def accum_store(buf, g):
            for nn in range(CH):
                for l in range(D // L):
                    v = buf[nn * k, pl.ds(l * L, L)]
                    for j in range(1, k):
                        v = v + buf[nn * k + j, pl.ds(l * L, L)]
                    acc[nn, pl.ds(l * L, L)] = v
            pltpu.sync_copy(acc, out_hbm.at[pl.ds(wid * NR + g * CH, CH)])

        gather(0, rows0, s0)

        def body(i, carry):
            g0 = 2 * i
            gather(g0 + 1, rows1, s1)
            drain(rows0, s0)
            accum_store(rows0, g0)

            @pl.when(g0 + 2 < NG)
            def _():
                gather(g0 + 2, rows0, s0)

            drain(rows1, s1)
            accum_store(rows1, g0 + 1)
            return carry

        lax.fori_loop(0, NG // 2, body, 0)

    return sc_gather


def _dense_body(xblk_ref, xs_ref, ef_ref, w1_ref, w2_ref, wef_ref, prm_ref,
                out_ref, *, inv_k):
    dot = functools.partial(lax.dot_general,
                            preferred_element_type=jnp.float32)
    mm = lambda a, bb: dot(a, bb, (((1,), (0,)), ((), ())))
    h = (mm(xblk_ref[...], w1_ref[...])
         + mm(xs_ref[...] * inv_k, w2_ref[...])
         + mm(ef_ref[...], wef_ref[...])
         + prm_ref[0:1, :])
    mu = jnp.mean(h, axis=1, keepdims=True)
    hc = h - mu
    var = jnp.mean(hc * hc, axis=1, keepdims=True)
    h = hc / jnp.sqrt(var + 1e-5) * prm_ref[1:2, :] + prm_ref[2:3, :]
    out_ref[...] = jnp.maximum(h, 0.0)


def kernel(x, pos, W, b, gamma, beta):
    n, D = x.shape
    k = min(_K, n - 1)
    C = ((n + 127) // 128) * 128
    bsz = next(bb for bb in (80, 40, 16, 8, 4, 2, 1) if n % bb == 0)

    pospad = jnp.zeros((C, 8), jnp.float32)
    pospad = pospad.at[:n, :3].set(pos).at[n:, :3].set(1e4)
    posT8 = pospad.T
    W1T = W[:, :D].T
    W2T = W[:, D:2 * D].T
    Wef = jnp.zeros((8, D), jnp.float32).at[:3].set(W[:, 2 * D:2 * D + 3].T)
    Wef = Wef.at[3].set(W[:, 2 * D + 3])
    prm = jnp.zeros((8, D), jnp.float32)
    prm = prm.at[0].set(b).at[1].set(gamma).at[2].set(beta)

    grid = (n // bsz,)
    full = lambda shp: pl.BlockSpec(shp, lambda i: (0, 0))
    blk = lambda shp: pl.BlockSpec(shp, lambda i: (i, 0))

    idx, ef8 = pl.pallas_call(
        functools.partial(_sel_body, bsz=bsz, n=n, k=k),
        grid=grid,
        in_specs=[full((8, C)), full((C, 8)), blk((bsz, 8))],
        out_specs=[blk((bsz, k)), blk((bsz, 8))],
        out_shape=[jax.ShapeDtypeStruct((n, k), jnp.int32),
                   jax.ShapeDtypeStruct((n, 8), jnp.float32)],
    )(posT8, pospad, pospad)

    NP = ((n + 511) // 512) * 512
    idxf = jnp.zeros((NP * k,), jnp.int32).at[:n * k].set(idx.reshape(-1))
    xsum = _make_sc_gather(NP, D, k)(x, idxf)

    return pl.pallas_call(
        functools.partial(_dense_body, inv_k=1.0 / k),
        grid=grid,
        in_specs=[blk((bsz, D)), blk((bsz, D)), blk((bsz, 8)),
                  full((D, D)), full((D, D)), full((8, D)), full((8, D))],
        out_specs=blk((bsz, D)),
        out_shape=jax.ShapeDtypeStruct((n, D), jnp.float32),
    )(x, xsum, ef8, W1T, W2T, Wef, prm)
